# baseline (speedup 1.0000x reference)
import functools

import jax
import jax.numpy as jnp
from jax import lax
from jax.experimental import pallas as pl
from jax.experimental.pallas import tpu as pltpu
from jax.experimental.pallas import tpu_sc as plsc

_C = 80
_Q = 900
_B = 128
_K = 300
_FLAT = _Q * _C
_CMAX = 384
_HBITS = 15
_HBINS = 1 << _HBITS
_SHIFT = 32 - _HBITS
_NWORKERS = 32
_ROWS_PER_W = _B // _NWORKERS
_CHUNKS = _FLAT // 16


def _sortable_key_u32(x_f32):
    b = lax.bitcast_convert_type(x_f32, jnp.int32)
    k = b ^ ((b >> 31) | jnp.int32(-(2 ** 31)))
    return lax.bitcast_convert_type(k, jnp.uint32)


def _sc_select_kernel(logits_hbm, ckey_hbm, cidx_hbm, rowbuf, hist, ckv, civ):
    wid = lax.axis_index("s") * 2 + lax.axis_index("c")

    def do_row(r, _):
        row = r * _NWORKERS + wid
        pltpu.sync_copy(logits_hbm.at[row], rowbuf)

        def zh(i, c):
            hist[pl.ds(i * 16, 16)] = jnp.zeros((16,), jnp.int32)
            return c
        lax.fori_loop(0, _HBINS // 16, zh, 0)

        def hp(i, c):
            ku = _sortable_key_u32(rowbuf[pl.ds(i * 16, 16)])
            binidx = lax.bitcast_convert_type(ku >> _SHIFT, jnp.int32)
            plsc.addupdate_scatter(hist, [binidx], jnp.ones((16,), jnp.int32))
            return c
        lax.fori_loop(0, _CHUNKS, hp, 0)

        def ts(i, carry):
            above, found, above_f = carry
            cc = _HBINS // 16 - 1 - i
            cnt = hist[pl.ds(cc * 16, 16)]
            cs = plsc.cumsum(cnt)
            tot = jnp.sum(cnt)
            suffix = tot - cs + cnt
            cond = (above + suffix) >= _K
            ntrue = jnp.max(plsc.all_reduce_population_count(cond))
            bstar = cc * 16 + ntrue - 1
            hit = (found < 0) & (ntrue > 0)
            lane_above = tot - jnp.max(
                jnp.where(lax.iota(jnp.int32, 16) == (ntrue - 1), cs, 0))
            found = jnp.where(hit, bstar, found)
            above_f = jnp.where(hit, above + lane_above, above_f)
            return above + tot, found, above_f
        _, bstar, above_f = lax.fori_loop(
            0, _HBINS // 16, ts,
            (jnp.int32(0), jnp.int32(-1), jnp.int32(0)))

        def zh2(i, c):
            hist[pl.ds(i * 16, 16)] = jnp.zeros((16,), jnp.int32)
            return c
        lax.fori_loop(0, 512 // 16, zh2, 0)
        bstar_u = bstar.astype(jnp.uint32)

        def hp2(i, c):
            ku = _sortable_key_u32(rowbuf[pl.ds(i * 16, 16)])
            m15 = (ku >> _SHIFT) == bstar_u
            sub = lax.bitcast_convert_type(
                (ku >> (_SHIFT - 9)) & jnp.uint32(511), jnp.int32)
            plsc.addupdate_scatter(hist, [sub], jnp.ones((16,), jnp.int32),
                                   mask=m15)
            return c
        lax.fori_loop(0, _CHUNKS, hp2, 0)

        def ts2(i, carry):
            above, found = carry
            cc = 512 // 16 - 1 - i
            cnt = hist[pl.ds(cc * 16, 16)]
            cs = plsc.cumsum(cnt)
            tot = jnp.sum(cnt)
            suffix = tot - cs + cnt
            cond = (above + suffix) >= _K
            ntrue = jnp.max(plsc.all_reduce_population_count(cond))
            sstar = cc * 16 + ntrue - 1
            found = jnp.where(found >= 0, found,
                              jnp.where(ntrue > 0, sstar, jnp.int32(-1)))
            return above + tot, found
        _, sstar = lax.fori_loop(0, 512 // 16, ts2,
                                 (above_f, jnp.int32(-1)))
        thr = lax.bitcast_convert_type(
            (bstar << _SHIFT) | (sstar << (_SHIFT - 9)), jnp.uint32)

        def ic(i, c):
            ckv[pl.ds(i * 16, 16)] = jnp.zeros((16,), jnp.uint32)
            civ[pl.ds(i * 16, 16)] = _FLAT + i * 16 + lax.iota(jnp.int32, 16)
            return c
        lax.fori_loop(0, _CMAX // 16, ic, 0)

        def cp(i, off):
            ku = _sortable_key_u32(rowbuf[pl.ds(i * 16, 16)])
            m = ku >= thr
            idx = i * 16 + lax.iota(jnp.int32, 16)
            plsc.store_compressed(ckv.at[pl.ds(off, 16)], ku, mask=m)
            plsc.store_compressed(civ.at[pl.ds(off, 16)], idx, mask=m)
            n = jnp.max(plsc.all_reduce_population_count(m))
            return jnp.minimum(off + n, jnp.int32(_CMAX - 16))
        lax.fori_loop(0, _CHUNKS, cp, jnp.int32(0))

        pltpu.sync_copy(ckv, ckey_hbm.at[row])
        pltpu.sync_copy(civ, cidx_hbm.at[row])
        return _
    lax.fori_loop(0, _ROWS_PER_W, do_row, 0)


def _sc_select(logits_2d):
    mesh = plsc.VectorSubcoreMesh(core_axis_name="c", subcore_axis_name="s")
    fn = functools.partial(
        pl.kernel,
        out_type=[
            jax.ShapeDtypeStruct((_B, _CMAX), jnp.uint32),
            jax.ShapeDtypeStruct((_B, _CMAX), jnp.int32),
        ],
        mesh=mesh,
        compiler_params=pltpu.CompilerParams(needs_layout_passes=False),
        scratch_types=[
            pltpu.VMEM((_FLAT,), jnp.float32),
            pltpu.VMEM((_HBINS,), jnp.int32),
            pltpu.VMEM((_CMAX,), jnp.uint32),
            pltpu.VMEM((_CMAX,), jnp.int32),
        ],
    )(_sc_select_kernel)
    return fn(logits_2d)


def _tc_finish_kernel(ckey_ref, cidx_ref, boxes_ref, layers_ref, sizes_ref,
                      lab_ref, box_ref, sc_ref, lay_ref):
    ku = ckey_ref[0, 0]
    ki = lax.bitcast_convert_type(ku, jnp.int32)
    idx = cidx_ref[0, 0]

    ch = 128
    nt = _CMAX // ch

    rank = jnp.zeros((_CMAX,), jnp.int32)
    for t in range(nt):
        kj = ku[t * ch:(t + 1) * ch]
        ij = idx[t * ch:(t + 1) * ch]
        gt = (kj[None, :] > ku[:, None]) | (
            (kj[None, :] == ku[:, None]) & (ij[None, :] < idx[:, None]))
        rank = rank + jnp.sum(gt.astype(jnp.int32), axis=1)
    m = rank < _K

    b4 = boxes_ref[0]
    cxy = b4[:, 0:2]
    wh = b4[:, 2:4]
    p1 = cxy - 0.5 * wh
    p2 = cxy + 0.5 * wh
    sz = sizes_ref[0, 0].astype(jnp.float32)
    scale4 = jnp.concatenate([sz, sz], axis=0)
    xyxy = jnp.concatenate([p1, p2], axis=1) * scale4[None, :]
    lay_f = layers_ref[0].astype(jnp.float32)
    vals = jnp.concatenate([xyxy, lay_f], axis=1)

    q = idx // _C
    gs = []
    for t in range(nt):
        qt = q[t * ch:(t + 1) * ch]
        oh = (qt[:, None] == lax.broadcasted_iota(jnp.int32, (ch, _Q), 1)
              ).astype(jnp.float32)
        gs.append(jnp.dot(oh, vals, preferred_element_type=jnp.float32,
                          precision=lax.Precision.HIGHEST))
    g = jnp.concatenate(gs, axis=0)
    glay = g[:, 4].astype(jnp.int32)

    comb = jnp.where(m, glay * 1024 + rank, jnp.int32(1 << 20) + idx)
    pos = jnp.zeros((_CMAX,), jnp.int32)
    for t in range(nt):
        cj = comb[t * ch:(t + 1) * ch]
        pos = pos + jnp.sum((cj[None, :] < comb[:, None]).astype(jnp.int32),
                            axis=1)

    bits = jnp.where(ki < 0, ki & jnp.int32(0x7FFFFFFF), ~ki)
    logit = lax.bitcast_convert_type(bits, jnp.float32)
    logit = jnp.where(m, logit, 0.0)
    score = 1.0 / (1.0 + jnp.exp(-logit))
    lab = (idx - q * _C).astype(jnp.float32)

    data = jnp.concatenate(
        [g[:, 0:4], score[:, None], lab[:, None], g[:, 4:5]], axis=1)

    out = jnp.zeros((_K, 7), jnp.float32)
    for t in range(nt):
        pt = pos[t * ch:(t + 1) * ch]
        mt = m[t * ch:(t + 1) * ch]
        permt = ((pt[None, :] == lax.broadcasted_iota(jnp.int32, (_K, ch), 0))
                 & mt[None, :]).astype(jnp.float32)
        out = out + jnp.dot(permt, data[t * ch:(t + 1) * ch],
                            preferred_element_type=jnp.float32,
                            precision=lax.Precision.HIGHEST)

    lab_ref[0, 0] = out[:, 5].astype(jnp.int32)
    box_ref[0] = out[:, 0:4]
    sc_ref[0, 0] = out[:, 4]
    lay_ref[0, 0] = out[:, 6].astype(jnp.int32)


def _tc_finish(ckey, cidx, pred_boxes, pred_layers, orig_sizes):
    labels, boxes, scores, layers = pl.pallas_call(
        _tc_finish_kernel,
        grid=(_B,),
        compiler_params=pltpu.CompilerParams(vmem_limit_bytes=100 * 1024 * 1024),
        in_specs=[
            pl.BlockSpec((1, 1, _CMAX), lambda b: (b, 0, 0)),
            pl.BlockSpec((1, 1, _CMAX), lambda b: (b, 0, 0)),
            pl.BlockSpec((1, _Q, 4), lambda b: (b, 0, 0)),
            pl.BlockSpec((1, _Q, 1), lambda b: (b, 0, 0)),
            pl.BlockSpec((1, 1, 2), lambda b: (b, 0, 0)),
        ],
        out_specs=[
            pl.BlockSpec((1, 1, _K), lambda b: (b, 0, 0)),
            pl.BlockSpec((1, _K, 4), lambda b: (b, 0, 0)),
            pl.BlockSpec((1, 1, _K), lambda b: (b, 0, 0)),
            pl.BlockSpec((1, 1, _K), lambda b: (b, 0, 0)),
        ],
        out_shape=[
            jax.ShapeDtypeStruct((_B, 1, _K), jnp.int32),
            jax.ShapeDtypeStruct((_B, _K, 4), jnp.float32),
            jax.ShapeDtypeStruct((_B, 1, _K), jnp.float32),
            jax.ShapeDtypeStruct((_B, 1, _K), jnp.int32),
        ],
    )(ckey.reshape(_B, 1, _CMAX), cidx.reshape(_B, 1, _CMAX),
      pred_boxes, pred_layers, orig_sizes.reshape(_B, 1, 2))
    return (labels.reshape(_B, _K), boxes, scores.reshape(_B, _K),
            layers.reshape(_B, _K))


def kernel(pred_logits, pred_boxes, pred_layers, orig_target_sizes):
    logits_2d = pred_logits.reshape(_B, _FLAT)
    ckey, cidx = _sc_select(logits_2d)
    labels, boxes, scores, layers = _tc_finish(
        ckey, cidx, pred_boxes, pred_layers, orig_target_sizes)
    return labels, boxes, scores, layers


# speedup vs baseline: 1.0266x; 1.0266x over previous
import functools

import jax
import jax.numpy as jnp
from jax import lax
from jax.experimental import pallas as pl
from jax.experimental.pallas import tpu as pltpu
from jax.experimental.pallas import tpu_sc as plsc

_C = 80
_Q = 900
_B = 128
_K = 300
_FLAT = _Q * _C
_CMAX = 384
_HBITS = 15
_HBINS = 1 << _HBITS
_SHIFT = 32 - _HBITS
_NWORKERS = 32
_ROWS_PER_W = _B // _NWORKERS
_CHUNKS = _FLAT // 16


def _sortable_key_u32(x_f32):
    b = lax.bitcast_convert_type(x_f32, jnp.int32)
    k = b ^ ((b >> 31) | jnp.int32(-(2 ** 31)))
    return lax.bitcast_convert_type(k, jnp.uint32)


def _sc_select_kernel(logits_hbm, ckey_hbm, cidx_hbm, rowbuf, hist, ckv, civ):
    wid = lax.axis_index("s") * 2 + lax.axis_index("c")

    def do_row(r, _):
        row = r * _NWORKERS + wid
        pltpu.sync_copy(logits_hbm.at[row], rowbuf)

        def zh(i, c):
            hist[pl.ds(i * 16, 16)] = jnp.zeros((16,), jnp.int32)
            return c
        lax.fori_loop(0, _HBINS // 16, zh, 0)

        def hp(i, c):
            ku = _sortable_key_u32(rowbuf[pl.ds(i * 16, 16)])
            binidx = lax.bitcast_convert_type(ku >> _SHIFT, jnp.int32)
            plsc.addupdate_scatter(hist, [binidx], jnp.ones((16,), jnp.int32))
            return c
        lax.fori_loop(0, _CHUNKS, hp, 0)

        def ts(i, carry):
            above, found, above_f = carry
            cc = _HBINS // 16 - 1 - i
            cnt = hist[pl.ds(cc * 16, 16)]
            cs = plsc.cumsum(cnt)
            tot = jnp.sum(cnt)
            suffix = tot - cs + cnt
            cond = (above + suffix) >= _K
            ntrue = jnp.max(plsc.all_reduce_population_count(cond))
            bstar = cc * 16 + ntrue - 1
            hit = (found < 0) & (ntrue > 0)
            lane_above = tot - jnp.max(
                jnp.where(lax.iota(jnp.int32, 16) == (ntrue - 1), cs, 0))
            found = jnp.where(hit, bstar, found)
            above_f = jnp.where(hit, above + lane_above, above_f)
            return above + tot, found, above_f
        _, bstar, above_f = lax.fori_loop(
            0, _HBINS // 16, ts,
            (jnp.int32(0), jnp.int32(-1), jnp.int32(0)))

        def zh2(i, c):
            hist[pl.ds(i * 16, 16)] = jnp.zeros((16,), jnp.int32)
            return c
        lax.fori_loop(0, 512 // 16, zh2, 0)
        bstar_u = bstar.astype(jnp.uint32)

        def hp2(i, c):
            ku = _sortable_key_u32(rowbuf[pl.ds(i * 16, 16)])
            m15 = (ku >> _SHIFT) == bstar_u
            sub = lax.bitcast_convert_type(
                (ku >> (_SHIFT - 9)) & jnp.uint32(511), jnp.int32)
            plsc.addupdate_scatter(hist, [sub], jnp.ones((16,), jnp.int32),
                                   mask=m15)
            return c
        lax.fori_loop(0, _CHUNKS, hp2, 0)

        def ts2(i, carry):
            above, found = carry
            cc = 512 // 16 - 1 - i
            cnt = hist[pl.ds(cc * 16, 16)]
            cs = plsc.cumsum(cnt)
            tot = jnp.sum(cnt)
            suffix = tot - cs + cnt
            cond = (above + suffix) >= _K
            ntrue = jnp.max(plsc.all_reduce_population_count(cond))
            sstar = cc * 16 + ntrue - 1
            found = jnp.where(found >= 0, found,
                              jnp.where(ntrue > 0, sstar, jnp.int32(-1)))
            return above + tot, found
        _, sstar = lax.fori_loop(0, 512 // 16, ts2,
                                 (above_f, jnp.int32(-1)))
        thr = lax.bitcast_convert_type(
            (bstar << _SHIFT) | (sstar << (_SHIFT - 9)), jnp.uint32)

        def ic(i, c):
            ckv[pl.ds(i * 16, 16)] = jnp.zeros((16,), jnp.uint32)
            civ[pl.ds(i * 16, 16)] = _FLAT + i * 16 + lax.iota(jnp.int32, 16)
            return c
        lax.fori_loop(0, _CMAX // 16, ic, 0)

        def cp(i, off):
            ku = _sortable_key_u32(rowbuf[pl.ds(i * 16, 16)])
            m = ku >= thr
            idx = i * 16 + lax.iota(jnp.int32, 16)
            plsc.store_compressed(ckv.at[pl.ds(off, 16)], ku, mask=m)
            plsc.store_compressed(civ.at[pl.ds(off, 16)], idx, mask=m)
            n = jnp.max(plsc.all_reduce_population_count(m))
            return jnp.minimum(off + n, jnp.int32(_CMAX - 16))
        lax.fori_loop(0, _CHUNKS, cp, jnp.int32(0))

        pltpu.sync_copy(ckv, ckey_hbm.at[row])
        pltpu.sync_copy(civ, cidx_hbm.at[row])
        return _
    lax.fori_loop(0, _ROWS_PER_W, do_row, 0)


def _sc_select(logits_2d):
    mesh = plsc.VectorSubcoreMesh(core_axis_name="c", subcore_axis_name="s")
    fn = functools.partial(
        pl.kernel,
        out_type=[
            jax.ShapeDtypeStruct((_B, _CMAX), jnp.uint32),
            jax.ShapeDtypeStruct((_B, _CMAX), jnp.int32),
        ],
        mesh=mesh,
        compiler_params=pltpu.CompilerParams(needs_layout_passes=False),
        scratch_types=[
            pltpu.VMEM((_FLAT,), jnp.float32),
            pltpu.VMEM((_HBINS,), jnp.int32),
            pltpu.VMEM((_CMAX,), jnp.uint32),
            pltpu.VMEM((_CMAX,), jnp.int32),
        ],
    )(_sc_select_kernel)
    return fn(logits_2d)


def _tc_finish_kernel(ckey_ref, cidx_ref, boxes_ref, layers_ref, sizes_ref,
                      lab_ref, box_ref, sc_ref, lay_ref):
    ku = ckey_ref[0, 0]
    ki = lax.bitcast_convert_type(ku, jnp.int32)
    idx = cidx_ref[0, 0]

    ch = 128
    nt = _CMAX // ch

    rank = jnp.zeros((_CMAX,), jnp.int32)
    for t in range(nt):
        kj = ku[t * ch:(t + 1) * ch]
        ij = idx[t * ch:(t + 1) * ch]
        gt = (kj[None, :] > ku[:, None]) | (
            (kj[None, :] == ku[:, None]) & (ij[None, :] < idx[:, None]))
        rank = rank + jnp.sum(gt.astype(jnp.int32), axis=1)
    m = rank < _K

    b4 = boxes_ref[0]
    cxy = b4[:, 0:2]
    wh = b4[:, 2:4]
    p1 = cxy - 0.5 * wh
    p2 = cxy + 0.5 * wh
    sz = sizes_ref[0, 0].astype(jnp.float32)
    scale4 = jnp.concatenate([sz, sz], axis=0)
    xyxy = jnp.concatenate([p1, p2], axis=1) * scale4[None, :]
    lay_f = layers_ref[0].astype(jnp.float32)
    vals = jnp.concatenate([xyxy, lay_f], axis=1)

    q = idx // _C
    gs = []
    for t in range(nt):
        qt = q[t * ch:(t + 1) * ch]
        oh = (qt[:, None] == lax.broadcasted_iota(jnp.int32, (ch, _Q), 1)
              ).astype(jnp.float32)
        gs.append(jnp.dot(oh, vals, preferred_element_type=jnp.float32))
    g = jnp.concatenate(gs, axis=0)
    glay = g[:, 4].astype(jnp.int32)

    comb = jnp.where(m, glay * 1024 + rank, jnp.int32(1 << 20) + idx)
    pos = jnp.zeros((_CMAX,), jnp.int32)
    for t in range(nt):
        cj = comb[t * ch:(t + 1) * ch]
        pos = pos + jnp.sum((cj[None, :] < comb[:, None]).astype(jnp.int32),
                            axis=1)

    bits = jnp.where(ki < 0, ki & jnp.int32(0x7FFFFFFF), ~ki)
    logit = lax.bitcast_convert_type(bits, jnp.float32)
    logit = jnp.where(m, logit, 0.0)
    score = 1.0 / (1.0 + jnp.exp(-logit))
    lab = (idx - q * _C).astype(jnp.float32)

    data = jnp.concatenate(
        [g[:, 0:4], score[:, None], lab[:, None], g[:, 4:5]], axis=1)

    out = jnp.zeros((_K, 7), jnp.float32)
    for t in range(nt):
        pt = pos[t * ch:(t + 1) * ch]
        mt = m[t * ch:(t + 1) * ch]
        permt = ((pt[None, :] == lax.broadcasted_iota(jnp.int32, (_K, ch), 0))
                 & mt[None, :]).astype(jnp.float32)
        out = out + jnp.dot(permt, data[t * ch:(t + 1) * ch],
                            preferred_element_type=jnp.float32)

    lab_ref[0, 0] = out[:, 5].astype(jnp.int32)
    box_ref[0] = out[:, 0:4]
    sc_ref[0, 0] = out[:, 4]
    lay_ref[0, 0] = out[:, 6].astype(jnp.int32)


def _tc_finish(ckey, cidx, pred_boxes, pred_layers, orig_sizes):
    labels, boxes, scores, layers = pl.pallas_call(
        _tc_finish_kernel,
        grid=(_B,),
        compiler_params=pltpu.CompilerParams(vmem_limit_bytes=100 * 1024 * 1024),
        in_specs=[
            pl.BlockSpec((1, 1, _CMAX), lambda b: (b, 0, 0)),
            pl.BlockSpec((1, 1, _CMAX), lambda b: (b, 0, 0)),
            pl.BlockSpec((1, _Q, 4), lambda b: (b, 0, 0)),
            pl.BlockSpec((1, _Q, 1), lambda b: (b, 0, 0)),
            pl.BlockSpec((1, 1, 2), lambda b: (b, 0, 0)),
        ],
        out_specs=[
            pl.BlockSpec((1, 1, _K), lambda b: (b, 0, 0)),
            pl.BlockSpec((1, _K, 4), lambda b: (b, 0, 0)),
            pl.BlockSpec((1, 1, _K), lambda b: (b, 0, 0)),
            pl.BlockSpec((1, 1, _K), lambda b: (b, 0, 0)),
        ],
        out_shape=[
            jax.ShapeDtypeStruct((_B, 1, _K), jnp.int32),
            jax.ShapeDtypeStruct((_B, _K, 4), jnp.float32),
            jax.ShapeDtypeStruct((_B, 1, _K), jnp.float32),
            jax.ShapeDtypeStruct((_B, 1, _K), jnp.int32),
        ],
    )(ckey.reshape(_B, 1, _CMAX), cidx.reshape(_B, 1, _CMAX),
      pred_boxes, pred_layers, orig_sizes.reshape(_B, 1, 2))
    return (labels.reshape(_B, _K), boxes, scores.reshape(_B, _K),
            layers.reshape(_B, _K))


def kernel(pred_logits, pred_boxes, pred_layers, orig_target_sizes):
    logits_2d = pred_logits.reshape(_B, _FLAT)
    ckey, cidx = _sc_select(logits_2d)
    labels, boxes, scores, layers = _tc_finish(
        ckey, cidx, pred_boxes, pred_layers, orig_target_sizes)
    return labels, boxes, scores, layers


# speedup vs baseline: 5.3377x; 5.1992x over previous
import functools

import jax
import jax.numpy as jnp
from jax import lax
from jax.experimental import pallas as pl
from jax.experimental.pallas import tpu as pltpu
from jax.experimental.pallas import tpu_sc as plsc

_C = 80
_Q = 900
_B = 128
_K = 300
_FLAT = _Q * _C
_CMAX = 384
_HBITS = 15
_HBINS = 1 << _HBITS
_SHIFT = 32 - _HBITS
_NWORKERS = 32
_ROWS_PER_W = _B // _NWORKERS
_CHUNKS = _FLAT // 16


def _sortable_key_u32(x_f32):
    b = lax.bitcast_convert_type(x_f32, jnp.int32)
    k = b ^ ((b >> 31) | jnp.int32(-(2 ** 31)))
    return lax.bitcast_convert_type(k, jnp.uint32)


def _sc_select_kernel(logits_hbm, ckey_hbm, cidx_hbm, rowbuf, hist, ckv, civ):
    wid = lax.axis_index("s") * 2 + lax.axis_index("c")

    def do_row(r, _):
        row = r * _NWORKERS + wid
        pltpu.sync_copy(logits_hbm.at[row], rowbuf)

        def zh(i, c):
            hist[pl.ds(i * 16, 16)] = jnp.zeros((16,), jnp.int32)
            return c
        lax.fori_loop(0, _HBINS // 16, zh, 0)

        def hp(i, c):
            ku = _sortable_key_u32(rowbuf[pl.ds(i * 16, 16)])
            binidx = lax.bitcast_convert_type(ku >> _SHIFT, jnp.int32)
            plsc.addupdate_scatter(hist, [binidx], jnp.ones((16,), jnp.int32))
            return c
        lax.fori_loop(0, _CHUNKS, hp, 0)

        def ts(i, carry):
            above, found, above_f = carry
            cc = _HBINS // 16 - 1 - i
            cnt = hist[pl.ds(cc * 16, 16)]
            cs = plsc.cumsum(cnt)
            tot = jnp.sum(cnt)
            suffix = tot - cs + cnt
            cond = (above + suffix) >= _K
            ntrue = jnp.max(plsc.all_reduce_population_count(cond))
            bstar = cc * 16 + ntrue - 1
            hit = (found < 0) & (ntrue > 0)
            lane_above = tot - jnp.max(
                jnp.where(lax.iota(jnp.int32, 16) == (ntrue - 1), cs, 0))
            found = jnp.where(hit, bstar, found)
            above_f = jnp.where(hit, above + lane_above, above_f)
            return above + tot, found, above_f
        _, bstar, above_f = lax.fori_loop(
            0, _HBINS // 16, ts,
            (jnp.int32(0), jnp.int32(-1), jnp.int32(0)))

        def zh2(i, c):
            hist[pl.ds(i * 16, 16)] = jnp.zeros((16,), jnp.int32)
            return c
        lax.fori_loop(0, 512 // 16, zh2, 0)
        bstar_u = bstar.astype(jnp.uint32)

        def hp2(i, c):
            ku = _sortable_key_u32(rowbuf[pl.ds(i * 16, 16)])
            m15 = (ku >> _SHIFT) == bstar_u
            sub = lax.bitcast_convert_type(
                (ku >> (_SHIFT - 9)) & jnp.uint32(511), jnp.int32)
            plsc.addupdate_scatter(hist, [sub], jnp.ones((16,), jnp.int32),
                                   mask=m15)
            return c
        lax.fori_loop(0, _CHUNKS, hp2, 0)

        def ts2(i, carry):
            above, found = carry
            cc = 512 // 16 - 1 - i
            cnt = hist[pl.ds(cc * 16, 16)]
            cs = plsc.cumsum(cnt)
            tot = jnp.sum(cnt)
            suffix = tot - cs + cnt
            cond = (above + suffix) >= _K
            ntrue = jnp.max(plsc.all_reduce_population_count(cond))
            sstar = cc * 16 + ntrue - 1
            found = jnp.where(found >= 0, found,
                              jnp.where(ntrue > 0, sstar, jnp.int32(-1)))
            return above + tot, found
        _, sstar = lax.fori_loop(0, 512 // 16, ts2,
                                 (above_f, jnp.int32(-1)))
        thr = lax.bitcast_convert_type(
            (bstar << _SHIFT) | (sstar << (_SHIFT - 9)), jnp.uint32)

        def ic(i, c):
            ckv[pl.ds(i * 16, 16)] = jnp.zeros((16,), jnp.uint32)
            civ[pl.ds(i * 16, 16)] = _FLAT + i * 16 + lax.iota(jnp.int32, 16)
            return c
        lax.fori_loop(0, _CMAX // 16, ic, 0)

        def cp(i, off):
            ku = _sortable_key_u32(rowbuf[pl.ds(i * 16, 16)])
            m = ku >= thr
            idx = i * 16 + lax.iota(jnp.int32, 16)
            plsc.store_compressed(ckv.at[pl.ds(off, 16)], ku, mask=m)
            plsc.store_compressed(civ.at[pl.ds(off, 16)], idx, mask=m)
            n = jnp.max(plsc.all_reduce_population_count(m))
            return jnp.minimum(off + n, jnp.int32(_CMAX - 16))
        lax.fori_loop(0, _CHUNKS, cp, jnp.int32(0))

        pltpu.sync_copy(ckv, ckey_hbm.at[row])
        pltpu.sync_copy(civ, cidx_hbm.at[row])
        return _
    lax.fori_loop(0, _ROWS_PER_W, do_row, 0)


def _sc_select(logits_2d):
    mesh = plsc.VectorSubcoreMesh(core_axis_name="c", subcore_axis_name="s")
    fn = functools.partial(
        pl.kernel,
        out_type=[
            jax.ShapeDtypeStruct((_B, _CMAX), jnp.uint32),
            jax.ShapeDtypeStruct((_B, _CMAX), jnp.int32),
        ],
        mesh=mesh,
        compiler_params=pltpu.CompilerParams(needs_layout_passes=False),
        scratch_types=[
            pltpu.VMEM((_FLAT,), jnp.float32),
            pltpu.VMEM((_HBINS,), jnp.int32),
            pltpu.VMEM((_CMAX,), jnp.uint32),
            pltpu.VMEM((_CMAX,), jnp.int32),
        ],
    )(_sc_select_kernel)
    return fn(logits_2d)


def _sc_finish_kernel(ckey_hbm, cidx_hbm, boxes_hbm, layers_hbm, scale_hbm,
                      lab_hbm, box_hbm, sc_hbm, lay_hbm,
                      ckv, civ, layrow, boxrow, sclrow,
                      rankb, clay, posb, olab, obox, osc, olay):
    wid = lax.axis_index("s") * 2 + lax.axis_index("c")
    nch = _CMAX // 16

    def do_row(r, _):
        row = r * _NWORKERS + wid
        pltpu.sync_copy(ckey_hbm.at[row], ckv.at[pl.ds(0, _CMAX)])
        pltpu.sync_copy(cidx_hbm.at[row], civ.at[pl.ds(0, _CMAX)])
        pltpu.sync_copy(layers_hbm.at[row], layrow)
        pltpu.sync_copy(boxes_hbm.at[row], boxrow)
        pltpu.sync_copy(scale_hbm.at[row], sclrow)

        def lg(c, _a):
            q = jnp.minimum(civ[pl.ds(c * 16, 16)] // _C, jnp.int32(_Q - 1))
            clay[pl.ds(c * 16, 16)] = plsc.load_gather(layrow, [q])
            return _a
        lax.fori_loop(0, nch, lg, 0)

        def rk(ci, _a):
            ki = ckv[pl.ds(ci * 16, 16)]
            ii = civ[pl.ds(ci * 16, 16)]

            def rj(j, racc):
                kj = ckv[pl.ds(j, 16)][0]
                ij = civ[pl.ds(j, 16)][0]
                gt = (kj > ki) | ((kj == ki) & (ij < ii))
                return racc + gt.astype(jnp.int32)
            racc = lax.fori_loop(0, _CMAX, rj, jnp.zeros((16,), jnp.int32))
            rankb[pl.ds(ci * 16, 16)] = racc
            return _a
        lax.fori_loop(0, nch, rk, 0)

        def cb(c, _a):
            rnk = rankb[pl.ds(c * 16, 16)]
            comb = jnp.where(rnk < _K,
                             clay[pl.ds(c * 16, 16)] * 1024 + rnk,
                             jnp.int32(1 << 20) + civ[pl.ds(c * 16, 16)])
            rankb[pl.ds(c * 16, 16)] = comb
            return _a
        lax.fori_loop(0, nch, cb, 0)

        def pr(ci, _a):
            ci_v = rankb[pl.ds(ci * 16, 16)]

            def pj(j, pacc):
                return pacc + (rankb[pl.ds(j, 16)][0] < ci_v).astype(jnp.int32)
            pacc = lax.fori_loop(0, _CMAX, pj, jnp.zeros((16,), jnp.int32))
            posb[pl.ds(ci * 16, 16)] = pacc
            return _a
        lax.fori_loop(0, nch, pr, 0)

        sclv = sclrow[pl.ds(0, 16)]
        s0 = sclv[0]
        s1 = sclv[1]

        def sc_out(c, _a):
            p = posb[pl.ds(c * 16, 16)]
            m = p < _K
            p = jnp.minimum(p, jnp.int32(_K - 1))
            idx = civ[pl.ds(c * 16, 16)]
            q = jnp.minimum(idx // _C, jnp.int32(_Q - 1))
            plsc.store_scatter(olab, [p], idx - q * _C, mask=m)
            plsc.store_scatter(olay, [p], clay[pl.ds(c * 16, 16)], mask=m)
            ki = lax.bitcast_convert_type(ckv[pl.ds(c * 16, 16)], jnp.int32)
            bits = jnp.where(ki < 0, ki & jnp.int32(0x7FFFFFFF), ~ki)
            x = lax.bitcast_convert_type(bits, jnp.float32)
            plsc.store_scatter(osc, [p], 1.0 / (1.0 + jnp.exp(-x)), mask=m)
            q4 = q * 4
            cx = plsc.load_gather(boxrow, [q4])
            cy = plsc.load_gather(boxrow, [q4 + 1])
            w = plsc.load_gather(boxrow, [q4 + 2])
            h = plsc.load_gather(boxrow, [q4 + 3])
            p4 = p * 4
            plsc.store_scatter(obox, [p4], (cx - 0.5 * w) * s0, mask=m)
            plsc.store_scatter(obox, [p4 + 1], (cy - 0.5 * h) * s1, mask=m)
            plsc.store_scatter(obox, [p4 + 2], (cx + 0.5 * w) * s0, mask=m)
            plsc.store_scatter(obox, [p4 + 3], (cy + 0.5 * h) * s1, mask=m)
            return _a
        lax.fori_loop(0, nch, sc_out, 0)

        pltpu.sync_copy(olab, lab_hbm.at[row])
        pltpu.sync_copy(obox, box_hbm.at[row])
        pltpu.sync_copy(osc, sc_hbm.at[row])
        pltpu.sync_copy(olay, lay_hbm.at[row])
        return _
    lax.fori_loop(0, _ROWS_PER_W, do_row, 0)


def _sc_finish(ckey, cidx, pred_boxes, pred_layers, orig_sizes):
    mesh = plsc.VectorSubcoreMesh(core_axis_name="c", subcore_axis_name="s")
    boxes_flat = pred_boxes.reshape(_B, _Q * 4)
    layers_p = jnp.pad(pred_layers.reshape(_B, _Q), ((0, 0), (0, 124)))
    sz = orig_sizes.astype(jnp.float32)
    scale = jnp.tile(sz, (1, 8))
    kp = 304
    fn = functools.partial(
        pl.kernel,
        out_type=[
            jax.ShapeDtypeStruct((_B, kp), jnp.int32),
            jax.ShapeDtypeStruct((_B, _K * 4), jnp.float32),
            jax.ShapeDtypeStruct((_B, kp), jnp.float32),
            jax.ShapeDtypeStruct((_B, kp), jnp.int32),
        ],
        mesh=mesh,
        compiler_params=pltpu.CompilerParams(needs_layout_passes=False),
        scratch_types=[
            pltpu.VMEM((_CMAX + 16,), jnp.uint32),
            pltpu.VMEM((_CMAX + 16,), jnp.int32),
            pltpu.VMEM((_Q + 124,), jnp.int32),
            pltpu.VMEM((_Q * 4,), jnp.float32),
            pltpu.VMEM((16,), jnp.float32),
            pltpu.VMEM((_CMAX + 16,), jnp.int32),
            pltpu.VMEM((_CMAX,), jnp.int32),
            pltpu.VMEM((_CMAX,), jnp.int32),
            pltpu.VMEM((kp,), jnp.int32),
            pltpu.VMEM((_K * 4,), jnp.float32),
            pltpu.VMEM((kp,), jnp.float32),
            pltpu.VMEM((kp,), jnp.int32),
        ],
    )(_sc_finish_kernel)
    olab, obox, osc, olay = fn(ckey, cidx, boxes_flat, layers_p, scale)
    return (olab[:, :_K], obox.reshape(_B, _K, 4), osc[:, :_K], olay[:, :_K])


def _tc_finish_kernel(ckey_ref, cidx_ref, boxes_ref, layers_ref, sizes_ref,
                      lab_ref, box_ref, sc_ref, lay_ref):
    ku = ckey_ref[0, 0]
    ki = lax.bitcast_convert_type(ku, jnp.int32)
    idx = cidx_ref[0, 0]

    ch = 128
    nt = _CMAX // ch

    rank = jnp.zeros((_CMAX,), jnp.int32)
    for t in range(nt):
        kj = ku[t * ch:(t + 1) * ch]
        ij = idx[t * ch:(t + 1) * ch]
        gt = (kj[None, :] > ku[:, None]) | (
            (kj[None, :] == ku[:, None]) & (ij[None, :] < idx[:, None]))
        rank = rank + jnp.sum(gt.astype(jnp.int32), axis=1)
    m = rank < _K

    b4 = boxes_ref[0]
    cxy = b4[:, 0:2]
    wh = b4[:, 2:4]
    p1 = cxy - 0.5 * wh
    p2 = cxy + 0.5 * wh
    sz = sizes_ref[0, 0].astype(jnp.float32)
    scale4 = jnp.concatenate([sz, sz], axis=0)
    xyxy = jnp.concatenate([p1, p2], axis=1) * scale4[None, :]
    lay_f = layers_ref[0].astype(jnp.float32)
    vals = jnp.concatenate([xyxy, lay_f], axis=1)

    q = idx // _C
    gs = []
    for t in range(nt):
        qt = q[t * ch:(t + 1) * ch]
        oh = (qt[:, None] == lax.broadcasted_iota(jnp.int32, (ch, _Q), 1)
              ).astype(jnp.float32)
        gs.append(jnp.dot(oh, vals, preferred_element_type=jnp.float32))
    g = jnp.concatenate(gs, axis=0)
    glay = g[:, 4].astype(jnp.int32)

    comb = jnp.where(m, glay * 1024 + rank, jnp.int32(1 << 20) + idx)
    pos = jnp.zeros((_CMAX,), jnp.int32)
    for t in range(nt):
        cj = comb[t * ch:(t + 1) * ch]
        pos = pos + jnp.sum((cj[None, :] < comb[:, None]).astype(jnp.int32),
                            axis=1)

    bits = jnp.where(ki < 0, ki & jnp.int32(0x7FFFFFFF), ~ki)
    logit = lax.bitcast_convert_type(bits, jnp.float32)
    logit = jnp.where(m, logit, 0.0)
    score = 1.0 / (1.0 + jnp.exp(-logit))
    lab = (idx - q * _C).astype(jnp.float32)

    data = jnp.concatenate(
        [g[:, 0:4], score[:, None], lab[:, None], g[:, 4:5]], axis=1)

    out = jnp.zeros((_K, 7), jnp.float32)
    for t in range(nt):
        pt = pos[t * ch:(t + 1) * ch]
        mt = m[t * ch:(t + 1) * ch]
        permt = ((pt[None, :] == lax.broadcasted_iota(jnp.int32, (_K, ch), 0))
                 & mt[None, :]).astype(jnp.float32)
        out = out + jnp.dot(permt, data[t * ch:(t + 1) * ch],
                            preferred_element_type=jnp.float32)

    lab_ref[0, 0] = out[:, 5].astype(jnp.int32)
    box_ref[0] = out[:, 0:4]
    sc_ref[0, 0] = out[:, 4]
    lay_ref[0, 0] = out[:, 6].astype(jnp.int32)


def _tc_finish(ckey, cidx, pred_boxes, pred_layers, orig_sizes):
    labels, boxes, scores, layers = pl.pallas_call(
        _tc_finish_kernel,
        grid=(_B,),
        compiler_params=pltpu.CompilerParams(vmem_limit_bytes=100 * 1024 * 1024),
        in_specs=[
            pl.BlockSpec((1, 1, _CMAX), lambda b: (b, 0, 0)),
            pl.BlockSpec((1, 1, _CMAX), lambda b: (b, 0, 0)),
            pl.BlockSpec((1, _Q, 4), lambda b: (b, 0, 0)),
            pl.BlockSpec((1, _Q, 1), lambda b: (b, 0, 0)),
            pl.BlockSpec((1, 1, 2), lambda b: (b, 0, 0)),
        ],
        out_specs=[
            pl.BlockSpec((1, 1, _K), lambda b: (b, 0, 0)),
            pl.BlockSpec((1, _K, 4), lambda b: (b, 0, 0)),
            pl.BlockSpec((1, 1, _K), lambda b: (b, 0, 0)),
            pl.BlockSpec((1, 1, _K), lambda b: (b, 0, 0)),
        ],
        out_shape=[
            jax.ShapeDtypeStruct((_B, 1, _K), jnp.int32),
            jax.ShapeDtypeStruct((_B, _K, 4), jnp.float32),
            jax.ShapeDtypeStruct((_B, 1, _K), jnp.float32),
            jax.ShapeDtypeStruct((_B, 1, _K), jnp.int32),
        ],
    )(ckey.reshape(_B, 1, _CMAX), cidx.reshape(_B, 1, _CMAX),
      pred_boxes, pred_layers, orig_sizes.reshape(_B, 1, 2))
    return (labels.reshape(_B, _K), boxes, scores.reshape(_B, _K),
            layers.reshape(_B, _K))


def kernel(pred_logits, pred_boxes, pred_layers, orig_target_sizes):
    logits_2d = pred_logits.reshape(_B, _FLAT)
    ckey, cidx = _sc_select(logits_2d)
    labels, boxes, scores, layers = _sc_finish(
        ckey, cidx, pred_boxes, pred_layers, orig_target_sizes)
    return labels, boxes, scores, layers


# speedup vs baseline: 5.3511x; 1.0025x over previous
import functools

import jax
import jax.numpy as jnp
from jax import lax
from jax.experimental import pallas as pl
from jax.experimental.pallas import tpu as pltpu
from jax.experimental.pallas import tpu_sc as plsc

_C = 80
_Q = 900
_B = 128
_K = 300
_FLAT = _Q * _C
_CMAX = 384
_HBITS = 15
_HBINS = 1 << _HBITS
_SHIFT = 32 - _HBITS
_NWORKERS = 32
_ROWS_PER_W = _B // _NWORKERS
_CHUNKS = _FLAT // 16


def _sortable_key_u32(x_f32):
    b = lax.bitcast_convert_type(x_f32, jnp.int32)
    k = b ^ ((b >> 31) | jnp.int32(-(2 ** 31)))
    return lax.bitcast_convert_type(k, jnp.uint32)


def _sc_all_kernel(logits_hbm, boxes_hbm, layers_hbm, scale_hbm,
                   lab_hbm, box_hbm, sc_hbm, lay_hbm,
                   rowbuf, hist, ckv, civ,
                   layrow, boxrow, sclrow, rankb, clay, posb,
                   olab, obox, osc, olay):
    wid = lax.axis_index("s") * 2 + lax.axis_index("c")
    nch = _CMAX // 16

    def do_row(r, _):
        row = r * _NWORKERS + wid
        pltpu.sync_copy(logits_hbm.at[row], rowbuf)
        pltpu.sync_copy(layers_hbm.at[row], layrow)
        pltpu.sync_copy(boxes_hbm.at[row], boxrow)
        pltpu.sync_copy(scale_hbm.at[row], sclrow)

        def zh(i, c):
            hist[pl.ds(i * 16, 16)] = jnp.zeros((16,), jnp.int32)
            return c
        lax.fori_loop(0, _HBINS // 16, zh, 0)

        def hp(i, c):
            ku = _sortable_key_u32(rowbuf[pl.ds(i * 16, 16)])
            binidx = lax.bitcast_convert_type(ku >> _SHIFT, jnp.int32)
            plsc.addupdate_scatter(hist, [binidx], jnp.ones((16,), jnp.int32))
            return c
        lax.fori_loop(0, _CHUNKS, hp, 0)

        def ts(i, carry):
            above, found, above_f = carry
            cc = _HBINS // 16 - 1 - i
            cnt = hist[pl.ds(cc * 16, 16)]
            cs = plsc.cumsum(cnt)
            tot = jnp.sum(cnt)
            suffix = tot - cs + cnt
            cond = (above + suffix) >= _K
            ntrue = jnp.max(plsc.all_reduce_population_count(cond))
            bstar = cc * 16 + ntrue - 1
            hit = (found < 0) & (ntrue > 0)
            lane_above = tot - jnp.max(
                jnp.where(lax.iota(jnp.int32, 16) == (ntrue - 1), cs, 0))
            found = jnp.where(hit, bstar, found)
            above_f = jnp.where(hit, above + lane_above, above_f)
            return above + tot, found, above_f
        _, bstar, above_f = lax.fori_loop(
            0, _HBINS // 16, ts,
            (jnp.int32(0), jnp.int32(-1), jnp.int32(0)))

        def zh2(i, c):
            hist[pl.ds(i * 16, 16)] = jnp.zeros((16,), jnp.int32)
            return c
        lax.fori_loop(0, 512 // 16, zh2, 0)
        bstar_u = bstar.astype(jnp.uint32)

        def hp2(i, c):
            ku = _sortable_key_u32(rowbuf[pl.ds(i * 16, 16)])
            m15 = (ku >> _SHIFT) == bstar_u
            sub = lax.bitcast_convert_type(
                (ku >> (_SHIFT - 9)) & jnp.uint32(511), jnp.int32)
            plsc.addupdate_scatter(hist, [sub], jnp.ones((16,), jnp.int32),
                                   mask=m15)
            return c
        lax.fori_loop(0, _CHUNKS, hp2, 0)

        def ts2(i, carry):
            above, found = carry
            cc = 512 // 16 - 1 - i
            cnt = hist[pl.ds(cc * 16, 16)]
            cs = plsc.cumsum(cnt)
            tot = jnp.sum(cnt)
            suffix = tot - cs + cnt
            cond = (above + suffix) >= _K
            ntrue = jnp.max(plsc.all_reduce_population_count(cond))
            sstar = cc * 16 + ntrue - 1
            found = jnp.where(found >= 0, found,
                              jnp.where(ntrue > 0, sstar, jnp.int32(-1)))
            return above + tot, found
        _, sstar = lax.fori_loop(0, 512 // 16, ts2,
                                 (above_f, jnp.int32(-1)))
        thr = lax.bitcast_convert_type(
            (bstar << _SHIFT) | (sstar << (_SHIFT - 9)), jnp.uint32)

        def ic(i, c):
            ckv[pl.ds(i * 16, 16)] = jnp.zeros((16,), jnp.uint32)
            civ[pl.ds(i * 16, 16)] = _FLAT + i * 16 + lax.iota(jnp.int32, 16)
            return c
        lax.fori_loop(0, _CMAX // 16, ic, 0)

        def cp(i, off):
            ku = _sortable_key_u32(rowbuf[pl.ds(i * 16, 16)])
            m = ku >= thr
            idx = i * 16 + lax.iota(jnp.int32, 16)
            plsc.store_compressed(ckv.at[pl.ds(off, 16)], ku, mask=m)
            plsc.store_compressed(civ.at[pl.ds(off, 16)], idx, mask=m)
            n = jnp.max(plsc.all_reduce_population_count(m))
            return jnp.minimum(off + n, jnp.int32(_CMAX - 16))
        lax.fori_loop(0, _CHUNKS, cp, jnp.int32(0))

        def lg(c, _a):
            qq = jnp.minimum(civ[pl.ds(c * 16, 16)] // _C, jnp.int32(_Q - 1))
            clay[pl.ds(c * 16, 16)] = plsc.load_gather(layrow, [qq])
            return _a
        lax.fori_loop(0, nch, lg, 0)

        def rk(ci, _a):
            ki = ckv[pl.ds(ci * 16, 16)]
            ii = civ[pl.ds(ci * 16, 16)]

            def rj(j, racc):
                kj = ckv[pl.ds(j, 16)][0]
                ij = civ[pl.ds(j, 16)][0]
                gt = (kj > ki) | ((kj == ki) & (ij < ii))
                return racc + gt.astype(jnp.int32)
            racc = lax.fori_loop(0, _CMAX, rj, jnp.zeros((16,), jnp.int32))
            rankb[pl.ds(ci * 16, 16)] = racc
            return _a
        lax.fori_loop(0, nch, rk, 0)

        def cb(c, _a):
            rnk = rankb[pl.ds(c * 16, 16)]
            comb = jnp.where(rnk < _K,
                             clay[pl.ds(c * 16, 16)] * 1024 + rnk,
                             jnp.int32(1 << 20) + civ[pl.ds(c * 16, 16)])
            rankb[pl.ds(c * 16, 16)] = comb
            return _a
        lax.fori_loop(0, nch, cb, 0)

        def pr(ci, _a):
            ci_v = rankb[pl.ds(ci * 16, 16)]

            def pj(j, pacc):
                return pacc + (rankb[pl.ds(j, 16)][0] < ci_v).astype(jnp.int32)
            pacc = lax.fori_loop(0, _CMAX, pj, jnp.zeros((16,), jnp.int32))
            posb[pl.ds(ci * 16, 16)] = pacc
            return _a
        lax.fori_loop(0, nch, pr, 0)

        sclv = sclrow[pl.ds(0, 16)]
        s0 = sclv[0]
        s1 = sclv[1]

        def sc_out(c, _a):
            p = posb[pl.ds(c * 16, 16)]
            m = p < _K
            p = jnp.minimum(p, jnp.int32(_K - 1))
            idx = civ[pl.ds(c * 16, 16)]
            qq = jnp.minimum(idx // _C, jnp.int32(_Q - 1))
            plsc.store_scatter(olab, [p], idx - qq * _C, mask=m)
            plsc.store_scatter(olay, [p], clay[pl.ds(c * 16, 16)], mask=m)
            ki = lax.bitcast_convert_type(ckv[pl.ds(c * 16, 16)], jnp.int32)
            bits = jnp.where(ki < 0, ki & jnp.int32(0x7FFFFFFF), ~ki)
            x = lax.bitcast_convert_type(bits, jnp.float32)
            plsc.store_scatter(osc, [p], 1.0 / (1.0 + jnp.exp(-x)), mask=m)
            q4 = qq * 4
            cx = plsc.load_gather(boxrow, [q4])
            cy = plsc.load_gather(boxrow, [q4 + 1])
            w = plsc.load_gather(boxrow, [q4 + 2])
            h = plsc.load_gather(boxrow, [q4 + 3])
            p4 = p * 4
            plsc.store_scatter(obox, [p4], (cx - 0.5 * w) * s0, mask=m)
            plsc.store_scatter(obox, [p4 + 1], (cy - 0.5 * h) * s1, mask=m)
            plsc.store_scatter(obox, [p4 + 2], (cx + 0.5 * w) * s0, mask=m)
            plsc.store_scatter(obox, [p4 + 3], (cy + 0.5 * h) * s1, mask=m)
            return _a
        lax.fori_loop(0, nch, sc_out, 0)

        pltpu.sync_copy(olab, lab_hbm.at[row])
        pltpu.sync_copy(obox, box_hbm.at[row])
        pltpu.sync_copy(osc, sc_hbm.at[row])
        pltpu.sync_copy(olay, lay_hbm.at[row])
        return _
    lax.fori_loop(0, _ROWS_PER_W, do_row, 0)


def _sc_all(logits_2d, pred_boxes, pred_layers, orig_sizes):
    mesh = plsc.VectorSubcoreMesh(core_axis_name="c", subcore_axis_name="s")
    boxes_flat = pred_boxes.reshape(_B, _Q * 4)
    layers_p = jnp.pad(pred_layers.reshape(_B, _Q), ((0, 0), (0, 124)))
    sz = orig_sizes.astype(jnp.float32)
    scale = jnp.tile(sz, (1, 8))
    kp = 304
    fn = functools.partial(
        pl.kernel,
        out_type=[
            jax.ShapeDtypeStruct((_B, kp), jnp.int32),
            jax.ShapeDtypeStruct((_B, _K * 4), jnp.float32),
            jax.ShapeDtypeStruct((_B, kp), jnp.float32),
            jax.ShapeDtypeStruct((_B, kp), jnp.int32),
        ],
        mesh=mesh,
        compiler_params=pltpu.CompilerParams(needs_layout_passes=False),
        scratch_types=[
            pltpu.VMEM((_FLAT,), jnp.float32),
            pltpu.VMEM((_HBINS,), jnp.int32),
            pltpu.VMEM((_CMAX + 16,), jnp.uint32),
            pltpu.VMEM((_CMAX + 16,), jnp.int32),
            pltpu.VMEM((_Q + 124,), jnp.int32),
            pltpu.VMEM((_Q * 4,), jnp.float32),
            pltpu.VMEM((16,), jnp.float32),
            pltpu.VMEM((_CMAX + 16,), jnp.int32),
            pltpu.VMEM((_CMAX,), jnp.int32),
            pltpu.VMEM((_CMAX,), jnp.int32),
            pltpu.VMEM((kp,), jnp.int32),
            pltpu.VMEM((_K * 4,), jnp.float32),
            pltpu.VMEM((kp,), jnp.float32),
            pltpu.VMEM((kp,), jnp.int32),
        ],
    )(_sc_all_kernel)
    olab, obox, osc, olay = fn(logits_2d, boxes_flat, layers_p, scale)
    return (olab[:, :_K], obox.reshape(_B, _K, 4), osc[:, :_K], olay[:, :_K])


def _sc_finish_kernel(ckey_hbm, cidx_hbm, boxes_hbm, layers_hbm, scale_hbm,
                      lab_hbm, box_hbm, sc_hbm, lay_hbm,
                      ckv, civ, layrow, boxrow, sclrow,
                      rankb, clay, posb, olab, obox, osc, olay):
    wid = lax.axis_index("s") * 2 + lax.axis_index("c")
    nch = _CMAX // 16

    def do_row(r, _):
        row = r * _NWORKERS + wid
        pltpu.sync_copy(ckey_hbm.at[row], ckv.at[pl.ds(0, _CMAX)])
        pltpu.sync_copy(cidx_hbm.at[row], civ.at[pl.ds(0, _CMAX)])
        pltpu.sync_copy(layers_hbm.at[row], layrow)
        pltpu.sync_copy(boxes_hbm.at[row], boxrow)
        pltpu.sync_copy(scale_hbm.at[row], sclrow)

        def lg(c, _a):
            q = jnp.minimum(civ[pl.ds(c * 16, 16)] // _C, jnp.int32(_Q - 1))
            clay[pl.ds(c * 16, 16)] = plsc.load_gather(layrow, [q])
            return _a
        lax.fori_loop(0, nch, lg, 0)

        def rk(ci, _a):
            ki = ckv[pl.ds(ci * 16, 16)]
            ii = civ[pl.ds(ci * 16, 16)]

            def rj(j, racc):
                kj = ckv[pl.ds(j, 16)][0]
                ij = civ[pl.ds(j, 16)][0]
                gt = (kj > ki) | ((kj == ki) & (ij < ii))
                return racc + gt.astype(jnp.int32)
            racc = lax.fori_loop(0, _CMAX, rj, jnp.zeros((16,), jnp.int32))
            rankb[pl.ds(ci * 16, 16)] = racc
            return _a
        lax.fori_loop(0, nch, rk, 0)

        def cb(c, _a):
            rnk = rankb[pl.ds(c * 16, 16)]
            comb = jnp.where(rnk < _K,
                             clay[pl.ds(c * 16, 16)] * 1024 + rnk,
                             jnp.int32(1 << 20) + civ[pl.ds(c * 16, 16)])
            rankb[pl.ds(c * 16, 16)] = comb
            return _a
        lax.fori_loop(0, nch, cb, 0)

        def pr(ci, _a):
            ci_v = rankb[pl.ds(ci * 16, 16)]

            def pj(j, pacc):
                return pacc + (rankb[pl.ds(j, 16)][0] < ci_v).astype(jnp.int32)
            pacc = lax.fori_loop(0, _CMAX, pj, jnp.zeros((16,), jnp.int32))
            posb[pl.ds(ci * 16, 16)] = pacc
            return _a
        lax.fori_loop(0, nch, pr, 0)

        sclv = sclrow[pl.ds(0, 16)]
        s0 = sclv[0]
        s1 = sclv[1]

        def sc_out(c, _a):
            p = posb[pl.ds(c * 16, 16)]
            m = p < _K
            p = jnp.minimum(p, jnp.int32(_K - 1))
            idx = civ[pl.ds(c * 16, 16)]
            q = jnp.minimum(idx // _C, jnp.int32(_Q - 1))
            plsc.store_scatter(olab, [p], idx - q * _C, mask=m)
            plsc.store_scatter(olay, [p], clay[pl.ds(c * 16, 16)], mask=m)
            ki = lax.bitcast_convert_type(ckv[pl.ds(c * 16, 16)], jnp.int32)
            bits = jnp.where(ki < 0, ki & jnp.int32(0x7FFFFFFF), ~ki)
            x = lax.bitcast_convert_type(bits, jnp.float32)
            plsc.store_scatter(osc, [p], 1.0 / (1.0 + jnp.exp(-x)), mask=m)
            q4 = q * 4
            cx = plsc.load_gather(boxrow, [q4])
            cy = plsc.load_gather(boxrow, [q4 + 1])
            w = plsc.load_gather(boxrow, [q4 + 2])
            h = plsc.load_gather(boxrow, [q4 + 3])
            p4 = p * 4
            plsc.store_scatter(obox, [p4], (cx - 0.5 * w) * s0, mask=m)
            plsc.store_scatter(obox, [p4 + 1], (cy - 0.5 * h) * s1, mask=m)
            plsc.store_scatter(obox, [p4 + 2], (cx + 0.5 * w) * s0, mask=m)
            plsc.store_scatter(obox, [p4 + 3], (cy + 0.5 * h) * s1, mask=m)
            return _a
        lax.fori_loop(0, nch, sc_out, 0)

        pltpu.sync_copy(olab, lab_hbm.at[row])
        pltpu.sync_copy(obox, box_hbm.at[row])
        pltpu.sync_copy(osc, sc_hbm.at[row])
        pltpu.sync_copy(olay, lay_hbm.at[row])
        return _
    lax.fori_loop(0, _ROWS_PER_W, do_row, 0)


def _sc_finish(ckey, cidx, pred_boxes, pred_layers, orig_sizes):
    mesh = plsc.VectorSubcoreMesh(core_axis_name="c", subcore_axis_name="s")
    boxes_flat = pred_boxes.reshape(_B, _Q * 4)
    layers_p = jnp.pad(pred_layers.reshape(_B, _Q), ((0, 0), (0, 124)))
    sz = orig_sizes.astype(jnp.float32)
    scale = jnp.tile(sz, (1, 8))
    kp = 304
    fn = functools.partial(
        pl.kernel,
        out_type=[
            jax.ShapeDtypeStruct((_B, kp), jnp.int32),
            jax.ShapeDtypeStruct((_B, _K * 4), jnp.float32),
            jax.ShapeDtypeStruct((_B, kp), jnp.float32),
            jax.ShapeDtypeStruct((_B, kp), jnp.int32),
        ],
        mesh=mesh,
        compiler_params=pltpu.CompilerParams(needs_layout_passes=False),
        scratch_types=[
            pltpu.VMEM((_CMAX + 16,), jnp.uint32),
            pltpu.VMEM((_CMAX + 16,), jnp.int32),
            pltpu.VMEM((_Q + 124,), jnp.int32),
            pltpu.VMEM((_Q * 4,), jnp.float32),
            pltpu.VMEM((16,), jnp.float32),
            pltpu.VMEM((_CMAX + 16,), jnp.int32),
            pltpu.VMEM((_CMAX,), jnp.int32),
            pltpu.VMEM((_CMAX,), jnp.int32),
            pltpu.VMEM((kp,), jnp.int32),
            pltpu.VMEM((_K * 4,), jnp.float32),
            pltpu.VMEM((kp,), jnp.float32),
            pltpu.VMEM((kp,), jnp.int32),
        ],
    )(_sc_finish_kernel)
    olab, obox, osc, olay = fn(ckey, cidx, boxes_flat, layers_p, scale)
    return (olab[:, :_K], obox.reshape(_B, _K, 4), osc[:, :_K], olay[:, :_K])


def _tc_finish_kernel(ckey_ref, cidx_ref, boxes_ref, layers_ref, sizes_ref,
                      lab_ref, box_ref, sc_ref, lay_ref):
    ku = ckey_ref[0, 0]
    ki = lax.bitcast_convert_type(ku, jnp.int32)
    idx = cidx_ref[0, 0]

    ch = 128
    nt = _CMAX // ch

    rank = jnp.zeros((_CMAX,), jnp.int32)
    for t in range(nt):
        kj = ku[t * ch:(t + 1) * ch]
        ij = idx[t * ch:(t + 1) * ch]
        gt = (kj[None, :] > ku[:, None]) | (
            (kj[None, :] == ku[:, None]) & (ij[None, :] < idx[:, None]))
        rank = rank + jnp.sum(gt.astype(jnp.int32), axis=1)
    m = rank < _K

    b4 = boxes_ref[0]
    cxy = b4[:, 0:2]
    wh = b4[:, 2:4]
    p1 = cxy - 0.5 * wh
    p2 = cxy + 0.5 * wh
    sz = sizes_ref[0, 0].astype(jnp.float32)
    scale4 = jnp.concatenate([sz, sz], axis=0)
    xyxy = jnp.concatenate([p1, p2], axis=1) * scale4[None, :]
    lay_f = layers_ref[0].astype(jnp.float32)
    vals = jnp.concatenate([xyxy, lay_f], axis=1)

    q = idx // _C
    gs = []
    for t in range(nt):
        qt = q[t * ch:(t + 1) * ch]
        oh = (qt[:, None] == lax.broadcasted_iota(jnp.int32, (ch, _Q), 1)
              ).astype(jnp.float32)
        gs.append(jnp.dot(oh, vals, preferred_element_type=jnp.float32))
    g = jnp.concatenate(gs, axis=0)
    glay = g[:, 4].astype(jnp.int32)

    comb = jnp.where(m, glay * 1024 + rank, jnp.int32(1 << 20) + idx)
    pos = jnp.zeros((_CMAX,), jnp.int32)
    for t in range(nt):
        cj = comb[t * ch:(t + 1) * ch]
        pos = pos + jnp.sum((cj[None, :] < comb[:, None]).astype(jnp.int32),
                            axis=1)

    bits = jnp.where(ki < 0, ki & jnp.int32(0x7FFFFFFF), ~ki)
    logit = lax.bitcast_convert_type(bits, jnp.float32)
    logit = jnp.where(m, logit, 0.0)
    score = 1.0 / (1.0 + jnp.exp(-logit))
    lab = (idx - q * _C).astype(jnp.float32)

    data = jnp.concatenate(
        [g[:, 0:4], score[:, None], lab[:, None], g[:, 4:5]], axis=1)

    out = jnp.zeros((_K, 7), jnp.float32)
    for t in range(nt):
        pt = pos[t * ch:(t + 1) * ch]
        mt = m[t * ch:(t + 1) * ch]
        permt = ((pt[None, :] == lax.broadcasted_iota(jnp.int32, (_K, ch), 0))
                 & mt[None, :]).astype(jnp.float32)
        out = out + jnp.dot(permt, data[t * ch:(t + 1) * ch],
                            preferred_element_type=jnp.float32)

    lab_ref[0, 0] = out[:, 5].astype(jnp.int32)
    box_ref[0] = out[:, 0:4]
    sc_ref[0, 0] = out[:, 4]
    lay_ref[0, 0] = out[:, 6].astype(jnp.int32)


def _tc_finish(ckey, cidx, pred_boxes, pred_layers, orig_sizes):
    labels, boxes, scores, layers = pl.pallas_call(
        _tc_finish_kernel,
        grid=(_B,),
        compiler_params=pltpu.CompilerParams(vmem_limit_bytes=100 * 1024 * 1024),
        in_specs=[
            pl.BlockSpec((1, 1, _CMAX), lambda b: (b, 0, 0)),
            pl.BlockSpec((1, 1, _CMAX), lambda b: (b, 0, 0)),
            pl.BlockSpec((1, _Q, 4), lambda b: (b, 0, 0)),
            pl.BlockSpec((1, _Q, 1), lambda b: (b, 0, 0)),
            pl.BlockSpec((1, 1, 2), lambda b: (b, 0, 0)),
        ],
        out_specs=[
            pl.BlockSpec((1, 1, _K), lambda b: (b, 0, 0)),
            pl.BlockSpec((1, _K, 4), lambda b: (b, 0, 0)),
            pl.BlockSpec((1, 1, _K), lambda b: (b, 0, 0)),
            pl.BlockSpec((1, 1, _K), lambda b: (b, 0, 0)),
        ],
        out_shape=[
            jax.ShapeDtypeStruct((_B, 1, _K), jnp.int32),
            jax.ShapeDtypeStruct((_B, _K, 4), jnp.float32),
            jax.ShapeDtypeStruct((_B, 1, _K), jnp.float32),
            jax.ShapeDtypeStruct((_B, 1, _K), jnp.int32),
        ],
    )(ckey.reshape(_B, 1, _CMAX), cidx.reshape(_B, 1, _CMAX),
      pred_boxes, pred_layers, orig_sizes.reshape(_B, 1, 2))
    return (labels.reshape(_B, _K), boxes, scores.reshape(_B, _K),
            layers.reshape(_B, _K))


def kernel(pred_logits, pred_boxes, pred_layers, orig_target_sizes):
    logits_2d = pred_logits.reshape(_B, _FLAT)
    labels, boxes, scores, layers = _sc_all(
        logits_2d, pred_boxes, pred_layers, orig_target_sizes)
    return labels, boxes, scores, layers
